# baseline (speedup 1.0000x reference)
import math

import jax
import jax.numpy as jnp
from jax import lax
from jax.experimental import pallas as pl
from jax.experimental.pallas import tpu as pltpu
from jax.experimental.pallas import tpu_sc as plsc

H = 4
DH = 32
D = 128
L = 2
N = 25000
E = 300000

NB = 1000
NACC = 25088
ROWS_PER_SUB = NACC // 16
EB = 64
CS = 3
BLOCKS_PER_SUB = 294
E_PAD = 16 * BLOCKS_PER_SUB * EB
AW = 72


def _lin_relu_body(x_ref, w_ref, b_ref, o_ref):
    y = jnp.dot(x_ref[0], w_ref[0], preferred_element_type=jnp.float32)
    o_ref[...] = jnp.maximum(y + b_ref[0, 0], 0.0)[None]


def _lin_relu(x2, w2, b2):
    return pl.pallas_call(
        _lin_relu_body,
        grid=(2, N // NB),
        in_specs=[
            pl.BlockSpec((1, NB, D), lambda t, i: (t, i, 0)),
            pl.BlockSpec((1, D, D), lambda t, i: (t, 0, 0)),
            pl.BlockSpec((1, 1, D), lambda t, i: (t, 0, 0)),
        ],
        out_specs=pl.BlockSpec((1, NB, D), lambda t, i: (t, i, 0)),
        out_shape=jax.ShapeDtypeStruct((2, N, D), jnp.float32),
    )(x2, w2, b2.reshape(2, 1, D))


def _proj_body(x_ref, w_ref, b_ref, o_ref):
    o_ref[...] = (
        jnp.dot(x_ref[...], w_ref[0, 0], preferred_element_type=jnp.float32)
        + b_ref[0, 0, 0]
    )


def _proj(x, wcat, bcat, p, w):
    return pl.pallas_call(
        _proj_body,
        grid=(N // NB, 2, p),
        in_specs=[
            pl.BlockSpec((NB, D), lambda i, j, q: (i, 0)),
            pl.BlockSpec((1, 1, D, w), lambda i, j, q: (q, j, 0, 0)),
            pl.BlockSpec((1, 1, 1, w), lambda i, j, q: (q, j, 0, 0)),
        ],
        out_specs=pl.BlockSpec(
            (NB, w), lambda i, j, q: (q * 2 * (N // NB) + j * (N // NB) + i, 0)),
        out_shape=jax.ShapeDtypeStruct((p * 2 * N, w), jnp.float32),
    )(x, wcat, bcat.reshape(p, 2, 1, w))


def _norm_agg(a):
    num = jnp.concatenate([a[0, :, 0:64], a[1, :, 0:64]], axis=1)
    den = jnp.concatenate(
        [
            jnp.broadcast_to(a[0, :, 64:65], (NB, DH)),
            jnp.broadcast_to(a[0, :, 65:66], (NB, DH)),
            jnp.broadcast_to(a[1, :, 64:65], (NB, DH)),
            jnp.broadcast_to(a[1, :, 65:66], (NB, DH)),
        ],
        axis=1,
    )
    return num / (den + 1e-16)


def _post_body(n_rel, acc_refs, x_ref, wa_ref, ba_ref, beta_ref, o_ref):
    agg = _norm_agg(acc_refs[0][...])
    for a_ref in acc_refs[1:]:
        agg = agg + _norm_agg(a_ref[...])
    o = jnp.dot(jax.nn.gelu(agg), wa_ref[...], preferred_element_type=jnp.float32)
    o = o + ba_ref[0]
    beta = beta_ref[0, 0]
    o_ref[...] = beta * o + (1.0 - beta) * x_ref[...]


def _post(accs, x_old, wa, ba, beta):
    n_rel = len(accs)

    def body(*refs):
        _post_body(n_rel, refs[:n_rel], *refs[n_rel:])

    return pl.pallas_call(
        body,
        grid=(N // NB,),
        in_specs=[pl.BlockSpec((2, NB, AW), lambda i: (0, i, 0))] * n_rel
        + [
            pl.BlockSpec((NB, D), lambda i: (i, 0)),
            pl.BlockSpec((D, D), lambda i: (0, 0)),
            pl.BlockSpec((1, D), lambda i: (0, 0)),
            pl.BlockSpec((1, 1), lambda i: (0, 0)),
        ],
        out_specs=pl.BlockSpec((NB, D), lambda i: (i, 0)),
        out_shape=jax.ShapeDtypeStruct((N, D), jnp.float32),
    )(*accs, x_old, wa.reshape(D, D), ba.reshape(1, D), beta.reshape(1, 1))


def _make_sc_agg(erow, kv_off):
    mesh = plsc.VectorSubcoreMesh(core_axis_name="c", subcore_axis_name="s")

    def body(src_hbm, dstg_hbm, dsts_hbm, q_hbm, kv_hbm, zro_hbm, out_hbm,
             ssrc, sdstg, sdsts, ikv, iq, isx, kvb, qb, wb, acc,
             sem_kv, sem_q):
        c = lax.axis_index("c")
        s = lax.axis_index("s")
        i16 = lax.iota(jnp.int32, 16)

        pltpu.sync_copy(
            zro_hbm.at[pl.ds(s * ROWS_PER_SUB, ROWS_PER_SUB)],
            acc.at[pl.ds(s * ROWS_PER_SUB, ROWS_PER_SUB)],
        )
        plsc.subcore_barrier()

        gdn = lax.GatherDimensionNumbers(
            offset_dims=(), collapsed_slice_dims=(0,), start_index_map=(0,))

        def allsum(v):
            for k in (8, 4, 2, 1):
                idx = lax.iota(jnp.int32, 16) ^ k
                v = v + lax.gather(v, idx[:, None], gdn, (1,),
                                   mode=lax.GatherScatterMode.PROMISE_IN_BOUNDS)
            return v

        def edge_body(e):
            q0 = qb[e, pl.ds(0, 16)]
            q1 = qb[e, pl.ds(16, 16)]
            q2 = qb[e, pl.ds(32, 16)]
            q3 = qb[e, pl.ds(48, 16)]
            k0 = kvb[e, pl.ds(0, 16)]
            k1 = kvb[e, pl.ds(16, 16)]
            k2 = kvb[e, pl.ds(32, 16)]
            k3 = kvb[e, pl.ds(48, 16)]
            ev0 = jnp.exp(allsum(q0 * k0 + q1 * k1))
            ev1 = jnp.exp(allsum(q2 * k2 + q3 * k3))
            wb[e, pl.ds(0, 16)] = ev0 * kvb[e, pl.ds(64, 16)]
            wb[e, pl.ds(16, 16)] = ev0 * kvb[e, pl.ds(80, 16)]
            wb[e, pl.ds(32, 16)] = ev1 * kvb[e, pl.ds(96, 16)]
            w3 = ev1 * kvb[e, pl.ds(112, 16)]
            wb[e, pl.ds(48, 16)] = w3
            sh = lax.gather(w3, ((i16 + 8) & 15)[:, None], gdn, (1,),
                            mode=lax.GatherScatterMode.PROMISE_IN_BOUNDS)
            tail = jnp.where(i16 == 8, ev0, jnp.where(i16 == 9, ev1, sh))
            wb[e, pl.ds(56, 16)] = tail

        base = erow * E_PAD + s * (BLOCKS_PER_SUB * EB)
        kv_c = kv_off * 2 * N + c * N
        q_c = c * N

        def chunk(ch, carry):
            coff = base + ch * (CS * EB)
            pltpu.sync_copy(src_hbm.at[pl.ds(coff, CS * EB)], ssrc)
            pltpu.sync_copy(dstg_hbm.at[pl.ds(coff, CS * EB)], sdstg)
            pltpu.sync_copy(dsts_hbm.at[pl.ds(coff, CS * EB)], sdsts)
            for b in range(CS):
                for j in range(EB // 16):
                    sl = pl.ds(b * EB + j * 16, 16)
                    dl = pl.ds(j * 16, 16)
                    ikv[dl] = ssrc[sl] + kv_c
                    iq[dl] = sdstg[sl] + q_c
                    isx[dl] = sdsts[sl]
                cp_kv = pltpu.async_copy(kv_hbm.at[ikv], kvb, sem_kv)
                cp_q = pltpu.async_copy(q_hbm.at[iq], qb, sem_q)
                cp_kv.wait()
                cp_q.wait()
                plsc.parallel_loop(0, EB, 1, unroll=4)(edge_body)
                pltpu.sync_copy(wb, acc.at[isx], add=True)
            return carry

        lax.fori_loop(0, BLOCKS_PER_SUB // CS, chunk, 0)

        plsc.subcore_barrier()
        pltpu.sync_copy(
            acc.at[pl.ds(s * ROWS_PER_SUB, ROWS_PER_SUB)],
            out_hbm.at[c, pl.ds(s * ROWS_PER_SUB, ROWS_PER_SUB)],
        )

    return pl.kernel(
        body,
        out_type=jax.ShapeDtypeStruct((2, NACC, AW), jnp.float32),
        mesh=mesh,
        compiler_params=pltpu.CompilerParams(use_tc_tiling_on_sc=False),
        scratch_types=[
            pltpu.VMEM((CS * EB,), jnp.int32),
            pltpu.VMEM((CS * EB,), jnp.int32),
            pltpu.VMEM((CS * EB,), jnp.int32),
            pltpu.VMEM((EB,), jnp.int32),
            pltpu.VMEM((EB,), jnp.int32),
            pltpu.VMEM((EB,), jnp.int32),
            pltpu.VMEM((EB, 2 * 64), jnp.float32),
            pltpu.VMEM((EB, 64), jnp.float32),
            pltpu.VMEM((EB, AW), jnp.float32),
            pltpu.VMEM_SHARED((NACC, AW), jnp.float32),
            pltpu.SemaphoreType.DMA,
            pltpu.SemaphoreType.DMA,
        ],
    )


_sc_writes = _make_sc_agg(0, 0)
_sc_cites = _make_sc_agg(1, 0)
_sc_rev = _make_sc_agg(2, 1)


def _blockdiag(mats):
    z = jnp.zeros((D, D), jnp.float32)
    for h in range(H):
        z = z.at[h * DH:(h + 1) * DH, h * DH:(h + 1) * DH].set(mats[h])
    return z


def _halves(w, b):
    w2 = w.shape[1]
    return (w.reshape(D, 2, w2 // 2).transpose(1, 0, 2),
            b.reshape(2, w2 // 2))


def _kv_halves(wk, bk_, wv, bv_):
    wh = [jnp.concatenate([wk[:, c * 64:(c + 1) * 64],
                           wv[:, c * 64:(c + 1) * 64]], axis=1) for c in (0, 1)]
    bh = [jnp.concatenate([bk_[c * 64:(c + 1) * 64],
                           bv_[c * 64:(c + 1) * 64]]) for c in (0, 1)]
    return jnp.stack(wh), jnp.stack(bh)


def kernel(x_paper, x_author, ei_writes, ei_cites, ei_rev, lin_in_W, lin_in_b,
           Wk, bk, Wq, bq, Wv, bv, Wa, ba, a_rel, m_rel, p_rel, skip):
    f32 = jnp.float32
    x_paper = x_paper.astype(f32)
    x_author = x_author.astype(f32)

    def pad_edges(ei):
        srcv = ei[0].astype(jnp.int32)
        dstv = ei[1].astype(jnp.int32)
        zpad = jnp.zeros((E_PAD - E,), jnp.int32)
        return (
            jnp.concatenate([srcv, zpad]),
            jnp.concatenate([dstv, zpad]),
            jnp.concatenate([dstv, jnp.full((E_PAD - E,), N, jnp.int32)]),
        )

    sw, gw, tw = pad_edges(ei_writes)
    sc_, gc, tc_ = pad_edges(ei_cites)
    sr, gr, tr = pad_edges(ei_rev)
    src_flat = jnp.concatenate([sw, sc_, sr])
    dstg_flat = jnp.concatenate([gw, gc, gr])
    dsts_flat = jnp.concatenate([tw, tc_, tr])

    xs = _lin_relu(
        jnp.stack([x_paper, x_author]),
        lin_in_W.astype(f32),
        lin_in_b.astype(f32),
    )
    xp, xa = xs[0], xs[1]

    scale = 1.0 / math.sqrt(DH)
    rel_src = (1, 0, 0)

    for l in range(L):
        wka, bka, wvm, bvm = [], [], [], []
        for r in range(3):
            st = rel_src[r]
            ablk = _blockdiag(a_rel[l, r] * (p_rel[l, r][:, None, None] * scale))
            mblk = _blockdiag(m_rel[l, r])
            wka.append(Wk[l, st] @ ablk)
            bka.append(bk[l, st] @ ablk)
            wvm.append(Wv[l, st] @ mblk)
            bvm.append(bv[l, st] @ mblk)

        qw_p, qb_p = _halves(Wq[l, 0], bq[l, 0])
        qw_a, qb_a = _halves(Wq[l, 1], bq[l, 1])
        kvw_c, kvb_c = _kv_halves(wka[1], bka[1], wvm[1], bvm[1])
        kvw_r, kvb_r = _kv_halves(wka[2], bka[2], wvm[2], bvm[2])
        kvw_w, kvb_w = _kv_halves(wka[0], bka[0], wvm[0], bvm[0])

        q_p = _proj(xp, qw_p[None], qb_p[None], 1, 64)
        q_a = _proj(xa, qw_a[None], qb_a[None], 1, 64)
        kv_p = _proj(xp, jnp.stack([kvw_c, kvw_r]), jnp.stack([kvb_c, kvb_r]),
                     2, 128)
        kv_a = _proj(xa, kvw_w[None], kvb_w[None], 1, 128)

        zro = jnp.zeros((NACC, AW), f32)
        acc_w = _sc_writes(src_flat, dstg_flat, dsts_flat, q_p, kv_a, zro)
        acc_c = _sc_cites(src_flat, dstg_flat, dsts_flat, q_p, kv_p, zro)
        acc_r = _sc_rev(src_flat, dstg_flat, dsts_flat, q_a, kv_p, zro)

        beta_p = jax.nn.sigmoid(skip[l, 0]).astype(f32)
        beta_a = jax.nn.sigmoid(skip[l, 1]).astype(f32)
        xp = _post([acc_w[:, :N], acc_c[:, :N]], xp, Wa[l, 0], ba[l, 0], beta_p)
        xa = _post([acc_r[:, :N]], xa, Wa[l, 1], ba[l, 1], beta_a)

    return xp, xa


# speedup vs baseline: 1.0091x; 1.0091x over previous
import math

import jax
import jax.numpy as jnp
from jax import lax
from jax.experimental import pallas as pl
from jax.experimental.pallas import tpu as pltpu
from jax.experimental.pallas import tpu_sc as plsc

H = 4
DH = 32
D = 128
L = 2
N = 25000
E = 300000

NB = 1000
NACC = 25088
ROWS_PER_SUB = NACC // 16
EB = 48
CS = 4
BLOCKS_PER_SUB = 392
E_PAD = 16 * BLOCKS_PER_SUB * EB
AW = 72


def _lin_relu_body(x_ref, w_ref, b_ref, o_ref):
    y = jnp.dot(x_ref[0], w_ref[0], preferred_element_type=jnp.float32)
    o_ref[...] = jnp.maximum(y + b_ref[0, 0], 0.0)[None]


def _lin_relu(x2, w2, b2):
    return pl.pallas_call(
        _lin_relu_body,
        grid=(2, N // NB),
        in_specs=[
            pl.BlockSpec((1, NB, D), lambda t, i: (t, i, 0)),
            pl.BlockSpec((1, D, D), lambda t, i: (t, 0, 0)),
            pl.BlockSpec((1, 1, D), lambda t, i: (t, 0, 0)),
        ],
        out_specs=pl.BlockSpec((1, NB, D), lambda t, i: (t, i, 0)),
        out_shape=jax.ShapeDtypeStruct((2, N, D), jnp.float32),
    )(x2, w2, b2.reshape(2, 1, D))


def _proj_body(x_ref, w_ref, b_ref, o_ref):
    o_ref[...] = (
        jnp.dot(x_ref[...], w_ref[0, 0], preferred_element_type=jnp.float32)
        + b_ref[0, 0, 0]
    )


def _proj(x, wcat, bcat, p, w):
    return pl.pallas_call(
        _proj_body,
        grid=(N // NB, 2, p),
        in_specs=[
            pl.BlockSpec((NB, D), lambda i, j, q: (i, 0)),
            pl.BlockSpec((1, 1, D, w), lambda i, j, q: (q, j, 0, 0)),
            pl.BlockSpec((1, 1, 1, w), lambda i, j, q: (q, j, 0, 0)),
        ],
        out_specs=pl.BlockSpec(
            (NB, w), lambda i, j, q: (q * 2 * (N // NB) + j * (N // NB) + i, 0)),
        out_shape=jax.ShapeDtypeStruct((p * 2 * N, w), jnp.float32),
    )(x, wcat, bcat.reshape(p, 2, 1, w))


def _norm_agg(a):
    num = jnp.concatenate([a[0, :, 0:64], a[1, :, 0:64]], axis=1)
    den = jnp.concatenate(
        [
            jnp.broadcast_to(a[0, :, 64:65], (NB, DH)),
            jnp.broadcast_to(a[0, :, 65:66], (NB, DH)),
            jnp.broadcast_to(a[1, :, 64:65], (NB, DH)),
            jnp.broadcast_to(a[1, :, 65:66], (NB, DH)),
        ],
        axis=1,
    )
    return num / (den + 1e-16)


def _post_body(n_rel, acc_refs, x_ref, wa_ref, ba_ref, beta_ref, o_ref):
    agg = _norm_agg(acc_refs[0][...])
    for a_ref in acc_refs[1:]:
        agg = agg + _norm_agg(a_ref[...])
    o = jnp.dot(jax.nn.gelu(agg), wa_ref[...], preferred_element_type=jnp.float32)
    o = o + ba_ref[0]
    beta = beta_ref[0, 0]
    o_ref[...] = beta * o + (1.0 - beta) * x_ref[...]


def _post(accs, x_old, wa, ba, beta):
    n_rel = len(accs)

    def body(*refs):
        _post_body(n_rel, refs[:n_rel], *refs[n_rel:])

    return pl.pallas_call(
        body,
        grid=(N // NB,),
        in_specs=[pl.BlockSpec((2, NB, AW), lambda i: (0, i, 0))] * n_rel
        + [
            pl.BlockSpec((NB, D), lambda i: (i, 0)),
            pl.BlockSpec((D, D), lambda i: (0, 0)),
            pl.BlockSpec((1, D), lambda i: (0, 0)),
            pl.BlockSpec((1, 1), lambda i: (0, 0)),
        ],
        out_specs=pl.BlockSpec((NB, D), lambda i: (i, 0)),
        out_shape=jax.ShapeDtypeStruct((N, D), jnp.float32),
    )(*accs, x_old, wa.reshape(D, D), ba.reshape(1, D), beta.reshape(1, 1))


def _make_sc_agg(erow, kv_off):
    mesh = plsc.VectorSubcoreMesh(core_axis_name="c", subcore_axis_name="s")

    def body(src_hbm, dstg_hbm, dsts_hbm, q_hbm, kv_hbm, zro_hbm, out_hbm,
             ssrc, sdstg, sdsts, ikv, iq, isx0, isx1, kvb, qb, wb0, wb1, acc,
             sem_kv, sem_q, sem_w0, sem_w1):
        isxs = (isx0, isx1)
        wbs = (wb0, wb1)
        sws = (sem_w0, sem_w1)
        c = lax.axis_index("c")
        s = lax.axis_index("s")
        i16 = lax.iota(jnp.int32, 16)

        pltpu.sync_copy(
            zro_hbm.at[pl.ds(s * ROWS_PER_SUB, ROWS_PER_SUB)],
            acc.at[pl.ds(s * ROWS_PER_SUB, ROWS_PER_SUB)],
        )
        plsc.subcore_barrier()

        gdn = lax.GatherDimensionNumbers(
            offset_dims=(), collapsed_slice_dims=(0,), start_index_map=(0,))

        def allsum(v):
            for k in (8, 4, 2, 1):
                idx = lax.iota(jnp.int32, 16) ^ k
                v = v + lax.gather(v, idx[:, None], gdn, (1,),
                                   mode=lax.GatherScatterMode.PROMISE_IN_BOUNDS)
            return v

        def edge_body(e, wb):
            q0 = qb[e, pl.ds(0, 16)]
            q1 = qb[e, pl.ds(16, 16)]
            q2 = qb[e, pl.ds(32, 16)]
            q3 = qb[e, pl.ds(48, 16)]
            k0 = kvb[e, pl.ds(0, 16)]
            k1 = kvb[e, pl.ds(16, 16)]
            k2 = kvb[e, pl.ds(32, 16)]
            k3 = kvb[e, pl.ds(48, 16)]
            ev0 = jnp.exp(allsum(q0 * k0 + q1 * k1))
            ev1 = jnp.exp(allsum(q2 * k2 + q3 * k3))
            wb[e, pl.ds(0, 16)] = ev0 * kvb[e, pl.ds(64, 16)]
            wb[e, pl.ds(16, 16)] = ev0 * kvb[e, pl.ds(80, 16)]
            wb[e, pl.ds(32, 16)] = ev1 * kvb[e, pl.ds(96, 16)]
            w3 = ev1 * kvb[e, pl.ds(112, 16)]
            wb[e, pl.ds(48, 16)] = w3
            sh = lax.gather(w3, ((i16 + 8) & 15)[:, None], gdn, (1,),
                            mode=lax.GatherScatterMode.PROMISE_IN_BOUNDS)
            tail = jnp.where(i16 == 8, ev0, jnp.where(i16 == 9, ev1, sh))
            wb[e, pl.ds(56, 16)] = tail

        base = erow * E_PAD + s * (BLOCKS_PER_SUB * EB)
        kv_c = kv_off * 2 * N + c * N
        q_c = c * N

        def chunk(ch, carry):
            coff = base + ch * (CS * EB)
            pltpu.sync_copy(src_hbm.at[pl.ds(coff, CS * EB)], ssrc)
            pltpu.sync_copy(dstg_hbm.at[pl.ds(coff, CS * EB)], sdstg)
            pltpu.sync_copy(dsts_hbm.at[pl.ds(coff, CS * EB)], sdsts)
            for b in range(CS):
                p = b % 2
                isx, wb, sw = isxs[p], wbs[p], sws[p]
                if b >= 2:
                    pltpu.make_async_copy(wb, acc.at[isx], sw).wait()
                else:
                    @pl.when(ch > 0)
                    def _(isx=isx, wb=wb, sw=sw):
                        pltpu.make_async_copy(wb, acc.at[isx], sw).wait()
                for j in range(EB // 16):
                    sl = pl.ds(b * EB + j * 16, 16)
                    dl = pl.ds(j * 16, 16)
                    ikv[dl] = ssrc[sl] + kv_c
                    iq[dl] = sdstg[sl] + q_c
                    isx[dl] = sdsts[sl]
                cp_kv = pltpu.async_copy(kv_hbm.at[ikv], kvb, sem_kv)
                cp_q = pltpu.async_copy(q_hbm.at[iq], qb, sem_q)
                cp_kv.wait()
                cp_q.wait()
                plsc.parallel_loop(0, EB, 1, unroll=4)(
                    lambda e, wb=wb: edge_body(e, wb))
                pltpu.async_copy(wb, acc.at[isx], sw, add=True)
            return carry

        lax.fori_loop(0, BLOCKS_PER_SUB // CS, chunk, 0)

        pltpu.make_async_copy(wb0, acc.at[isx0], sem_w0).wait()
        pltpu.make_async_copy(wb1, acc.at[isx1], sem_w1).wait()
        plsc.subcore_barrier()
        pltpu.sync_copy(
            acc.at[pl.ds(s * ROWS_PER_SUB, ROWS_PER_SUB)],
            out_hbm.at[c, pl.ds(s * ROWS_PER_SUB, ROWS_PER_SUB)],
        )

    return pl.kernel(
        body,
        out_type=jax.ShapeDtypeStruct((2, NACC, AW), jnp.float32),
        mesh=mesh,
        compiler_params=pltpu.CompilerParams(use_tc_tiling_on_sc=False),
        scratch_types=[
            pltpu.VMEM((CS * EB,), jnp.int32),
            pltpu.VMEM((CS * EB,), jnp.int32),
            pltpu.VMEM((CS * EB,), jnp.int32),
            pltpu.VMEM((EB,), jnp.int32),
            pltpu.VMEM((EB,), jnp.int32),
            pltpu.VMEM((EB,), jnp.int32),
            pltpu.VMEM((EB,), jnp.int32),
            pltpu.VMEM((EB, 2 * 64), jnp.float32),
            pltpu.VMEM((EB, 64), jnp.float32),
            pltpu.VMEM((EB, AW), jnp.float32),
            pltpu.VMEM((EB, AW), jnp.float32),
            pltpu.VMEM_SHARED((NACC, AW), jnp.float32),
            pltpu.SemaphoreType.DMA,
            pltpu.SemaphoreType.DMA,
            pltpu.SemaphoreType.DMA,
            pltpu.SemaphoreType.DMA,
        ],
    )


_sc_writes = _make_sc_agg(0, 0)
_sc_cites = _make_sc_agg(1, 0)
_sc_rev = _make_sc_agg(2, 1)


def _blockdiag(mats):
    z = jnp.zeros((D, D), jnp.float32)
    for h in range(H):
        z = z.at[h * DH:(h + 1) * DH, h * DH:(h + 1) * DH].set(mats[h])
    return z


def _halves(w, b):
    w2 = w.shape[1]
    return (w.reshape(D, 2, w2 // 2).transpose(1, 0, 2),
            b.reshape(2, w2 // 2))


def _kv_halves(wk, bk_, wv, bv_):
    wh = [jnp.concatenate([wk[:, c * 64:(c + 1) * 64],
                           wv[:, c * 64:(c + 1) * 64]], axis=1) for c in (0, 1)]
    bh = [jnp.concatenate([bk_[c * 64:(c + 1) * 64],
                           bv_[c * 64:(c + 1) * 64]]) for c in (0, 1)]
    return jnp.stack(wh), jnp.stack(bh)


def kernel(x_paper, x_author, ei_writes, ei_cites, ei_rev, lin_in_W, lin_in_b,
           Wk, bk, Wq, bq, Wv, bv, Wa, ba, a_rel, m_rel, p_rel, skip):
    f32 = jnp.float32
    x_paper = x_paper.astype(f32)
    x_author = x_author.astype(f32)

    def pad_edges(ei):
        srcv = ei[0].astype(jnp.int32)
        dstv = ei[1].astype(jnp.int32)
        zpad = jnp.zeros((E_PAD - E,), jnp.int32)
        return (
            jnp.concatenate([srcv, zpad]),
            jnp.concatenate([dstv, zpad]),
            jnp.concatenate([dstv, jnp.full((E_PAD - E,), N, jnp.int32)]),
        )

    sw, gw, tw = pad_edges(ei_writes)
    sc_, gc, tc_ = pad_edges(ei_cites)
    sr, gr, tr = pad_edges(ei_rev)
    src_flat = jnp.concatenate([sw, sc_, sr])
    dstg_flat = jnp.concatenate([gw, gc, gr])
    dsts_flat = jnp.concatenate([tw, tc_, tr])

    xs = _lin_relu(
        jnp.stack([x_paper, x_author]),
        lin_in_W.astype(f32),
        lin_in_b.astype(f32),
    )
    xp, xa = xs[0], xs[1]

    scale = 1.0 / math.sqrt(DH)
    rel_src = (1, 0, 0)

    for l in range(L):
        wka, bka, wvm, bvm = [], [], [], []
        for r in range(3):
            st = rel_src[r]
            ablk = _blockdiag(a_rel[l, r] * (p_rel[l, r][:, None, None] * scale))
            mblk = _blockdiag(m_rel[l, r])
            wka.append(Wk[l, st] @ ablk)
            bka.append(bk[l, st] @ ablk)
            wvm.append(Wv[l, st] @ mblk)
            bvm.append(bv[l, st] @ mblk)

        qw_p, qb_p = _halves(Wq[l, 0], bq[l, 0])
        qw_a, qb_a = _halves(Wq[l, 1], bq[l, 1])
        kvw_c, kvb_c = _kv_halves(wka[1], bka[1], wvm[1], bvm[1])
        kvw_r, kvb_r = _kv_halves(wka[2], bka[2], wvm[2], bvm[2])
        kvw_w, kvb_w = _kv_halves(wka[0], bka[0], wvm[0], bvm[0])

        q_p = _proj(xp, qw_p[None], qb_p[None], 1, 64)
        q_a = _proj(xa, qw_a[None], qb_a[None], 1, 64)
        kv_p = _proj(xp, jnp.stack([kvw_c, kvw_r]), jnp.stack([kvb_c, kvb_r]),
                     2, 128)
        kv_a = _proj(xa, kvw_w[None], kvb_w[None], 1, 128)

        zro = jnp.zeros((NACC, AW), f32)
        acc_w = _sc_writes(src_flat, dstg_flat, dsts_flat, q_p, kv_a, zro)
        acc_c = _sc_cites(src_flat, dstg_flat, dsts_flat, q_p, kv_p, zro)
        acc_r = _sc_rev(src_flat, dstg_flat, dsts_flat, q_a, kv_p, zro)

        beta_p = jax.nn.sigmoid(skip[l, 0]).astype(f32)
        beta_a = jax.nn.sigmoid(skip[l, 1]).astype(f32)
        xp = _post([acc_w[:, :N], acc_c[:, :N]], xp, Wa[l, 0], ba[l, 0], beta_p)
        xa = _post([acc_r[:, :N]], xa, Wa[l, 1], ba[l, 1], beta_a)

    return xp, xa


# speedup vs baseline: 1.0936x; 1.0837x over previous
import math

import jax
import jax.numpy as jnp
from jax import lax
from jax.experimental import pallas as pl
from jax.experimental.pallas import tpu as pltpu
from jax.experimental.pallas import tpu_sc as plsc

H = 4
DH = 32
D = 128
L = 2
N = 25000
E = 300000

NB = 1000
NACC = 25088
ROWS_PER_SUB = NACC // 16
EB = 32
CS = 6
BLOCKS_PER_SUB = 588
E_PAD = 16 * BLOCKS_PER_SUB * EB
AW = 72


def _lin_relu_body(x_ref, w_ref, b_ref, o_ref):
    y = jnp.dot(x_ref[0], w_ref[0], preferred_element_type=jnp.float32)
    o_ref[...] = jnp.maximum(y + b_ref[0, 0], 0.0)[None]


def _lin_relu(x2, w2, b2):
    return pl.pallas_call(
        _lin_relu_body,
        grid=(2, N // NB),
        in_specs=[
            pl.BlockSpec((1, NB, D), lambda t, i: (t, i, 0)),
            pl.BlockSpec((1, D, D), lambda t, i: (t, 0, 0)),
            pl.BlockSpec((1, 1, D), lambda t, i: (t, 0, 0)),
        ],
        out_specs=pl.BlockSpec((1, NB, D), lambda t, i: (t, i, 0)),
        out_shape=jax.ShapeDtypeStruct((2, N, D), jnp.float32),
    )(x2, w2, b2.reshape(2, 1, D))


def _proj_body(x_ref, w_ref, b_ref, o_ref):
    o_ref[...] = (
        jnp.dot(x_ref[...], w_ref[0, 0], preferred_element_type=jnp.float32)
        + b_ref[0, 0, 0]
    )


def _proj(x, wcat, bcat, p, w):
    return pl.pallas_call(
        _proj_body,
        grid=(N // NB, 2, p),
        in_specs=[
            pl.BlockSpec((NB, D), lambda i, j, q: (i, 0)),
            pl.BlockSpec((1, 1, D, w), lambda i, j, q: (q, j, 0, 0)),
            pl.BlockSpec((1, 1, 1, w), lambda i, j, q: (q, j, 0, 0)),
        ],
        out_specs=pl.BlockSpec(
            (NB, w), lambda i, j, q: (q * 2 * (N // NB) + j * (N // NB) + i, 0)),
        out_shape=jax.ShapeDtypeStruct((p * 2 * N, w), jnp.float32),
    )(x, wcat, bcat.reshape(p, 2, 1, w))


def _norm_agg(a):
    num = jnp.concatenate([a[0, :, 0:64], a[1, :, 0:64]], axis=1)
    den = jnp.concatenate(
        [
            jnp.broadcast_to(a[0, :, 64:65], (NB, DH)),
            jnp.broadcast_to(a[0, :, 65:66], (NB, DH)),
            jnp.broadcast_to(a[1, :, 64:65], (NB, DH)),
            jnp.broadcast_to(a[1, :, 65:66], (NB, DH)),
        ],
        axis=1,
    )
    return num / (den + 1e-16)


def _post_body(n_rel, acc_refs, x_ref, wa_ref, ba_ref, beta_ref, o_ref):
    agg = _norm_agg(acc_refs[0][...])
    for a_ref in acc_refs[1:]:
        agg = agg + _norm_agg(a_ref[...])
    o = jnp.dot(jax.nn.gelu(agg), wa_ref[...], preferred_element_type=jnp.float32)
    o = o + ba_ref[0]
    beta = beta_ref[0, 0]
    o_ref[...] = beta * o + (1.0 - beta) * x_ref[...]


def _post(accs, x_old, wa, ba, beta):
    n_rel = len(accs)

    def body(*refs):
        _post_body(n_rel, refs[:n_rel], *refs[n_rel:])

    return pl.pallas_call(
        body,
        grid=(N // NB,),
        in_specs=[pl.BlockSpec((2, NB, AW), lambda i: (0, i, 0))] * n_rel
        + [
            pl.BlockSpec((NB, D), lambda i: (i, 0)),
            pl.BlockSpec((D, D), lambda i: (0, 0)),
            pl.BlockSpec((1, D), lambda i: (0, 0)),
            pl.BlockSpec((1, 1), lambda i: (0, 0)),
        ],
        out_specs=pl.BlockSpec((NB, D), lambda i: (i, 0)),
        out_shape=jax.ShapeDtypeStruct((N, D), jnp.float32),
    )(*accs, x_old, wa.reshape(D, D), ba.reshape(1, D), beta.reshape(1, 1))


def _make_sc_agg(erow, kv_off):
    mesh = plsc.VectorSubcoreMesh(core_axis_name="c", subcore_axis_name="s")

    def body(src_hbm, dstg_hbm, dsts_hbm, q_hbm, kv_hbm, zro_hbm, out_hbm,
             ssrc, sdstg, sdsts, ikv0, ikv1, iq0, iq1, isx0, isx1,
             kvb0, kvb1, qb0, qb1, wb0, wb1, acc,
             sem_kv0, sem_kv1, sem_q0, sem_q1, sem_w0, sem_w1):
        sets = (
            (ikv0, iq0, isx0, kvb0, qb0, wb0, sem_kv0, sem_q0, sem_w0),
            (ikv1, iq1, isx1, kvb1, qb1, wb1, sem_kv1, sem_q1, sem_w1),
        )
        c = lax.axis_index("c")
        s = lax.axis_index("s")
        i16 = lax.iota(jnp.int32, 16)

        pltpu.sync_copy(
            zro_hbm.at[pl.ds(s * ROWS_PER_SUB, ROWS_PER_SUB)],
            acc.at[pl.ds(s * ROWS_PER_SUB, ROWS_PER_SUB)],
        )
        plsc.subcore_barrier()

        gdn = lax.GatherDimensionNumbers(
            offset_dims=(), collapsed_slice_dims=(0,), start_index_map=(0,))

        def allsum(v):
            for k in (8, 4, 2, 1):
                idx = lax.iota(jnp.int32, 16) ^ k
                v = v + lax.gather(v, idx[:, None], gdn, (1,),
                                   mode=lax.GatherScatterMode.PROMISE_IN_BOUNDS)
            return v

        def edge_body(e, kvb, qb, wb):
            q0 = qb[e, pl.ds(0, 16)]
            q1 = qb[e, pl.ds(16, 16)]
            q2 = qb[e, pl.ds(32, 16)]
            q3 = qb[e, pl.ds(48, 16)]
            k0 = kvb[e, pl.ds(0, 16)]
            k1 = kvb[e, pl.ds(16, 16)]
            k2 = kvb[e, pl.ds(32, 16)]
            k3 = kvb[e, pl.ds(48, 16)]
            ev0 = jnp.exp(allsum(q0 * k0 + q1 * k1))
            ev1 = jnp.exp(allsum(q2 * k2 + q3 * k3))
            wb[e, pl.ds(0, 16)] = ev0 * kvb[e, pl.ds(64, 16)]
            wb[e, pl.ds(16, 16)] = ev0 * kvb[e, pl.ds(80, 16)]
            wb[e, pl.ds(32, 16)] = ev1 * kvb[e, pl.ds(96, 16)]
            w3 = ev1 * kvb[e, pl.ds(112, 16)]
            wb[e, pl.ds(48, 16)] = w3
            sh = lax.gather(w3, ((i16 + 8) & 15)[:, None], gdn, (1,),
                            mode=lax.GatherScatterMode.PROMISE_IN_BOUNDS)
            tail = jnp.where(i16 == 8, ev0, jnp.where(i16 == 9, ev1, sh))
            wb[e, pl.ds(56, 16)] = tail

        base = erow * E_PAD + s * (BLOCKS_PER_SUB * EB)
        kv_c = kv_off * 2 * N + c * N
        q_c = c * N

        def stage_and_gather(b, S):
            ikv, iq, isx = S[0], S[1], S[2]
            for j in range(EB // 16):
                sl = pl.ds(b * EB + j * 16, 16)
                dl = pl.ds(j * 16, 16)
                ikv[dl] = ssrc[sl] + kv_c
                iq[dl] = sdstg[sl] + q_c
                isx[dl] = sdsts[sl]
            pltpu.async_copy(kv_hbm.at[ikv], S[3], S[6])
            pltpu.async_copy(q_hbm.at[iq], S[4], S[7])

        def wait_scatter(S):
            pltpu.make_async_copy(S[5], acc.at[S[2]], S[8]).wait()

        def chunk(ch, carry):
            coff = base + ch * (CS * EB)
            pltpu.sync_copy(src_hbm.at[pl.ds(coff, CS * EB)], ssrc)
            pltpu.sync_copy(dstg_hbm.at[pl.ds(coff, CS * EB)], sdstg)
            pltpu.sync_copy(dsts_hbm.at[pl.ds(coff, CS * EB)], sdsts)

            @pl.when(ch > 0)
            def _():
                wait_scatter(sets[0])

            stage_and_gather(0, sets[0])
            for b in range(CS):
                p = b % 2
                S = sets[p]
                if b + 1 < CS:
                    Q = sets[1 - p]
                    if b + 1 >= 2:
                        wait_scatter(Q)
                    else:
                        @pl.when(ch > 0)
                        def _(Q=Q):
                            wait_scatter(Q)
                    stage_and_gather(b + 1, Q)
                pltpu.make_async_copy(kv_hbm.at[S[0]], S[3], S[6]).wait()
                pltpu.make_async_copy(q_hbm.at[S[1]], S[4], S[7]).wait()
                plsc.parallel_loop(0, EB, 1, unroll=4)(
                    lambda e, S=S: edge_body(e, S[3], S[4], S[5]))
                pltpu.async_copy(S[5], acc.at[S[2]], S[8], add=True)
            return carry

        lax.fori_loop(0, BLOCKS_PER_SUB // CS, chunk, 0)

        wait_scatter(sets[0])
        wait_scatter(sets[1])
        plsc.subcore_barrier()
        pltpu.sync_copy(
            acc.at[pl.ds(s * ROWS_PER_SUB, ROWS_PER_SUB)],
            out_hbm.at[c, pl.ds(s * ROWS_PER_SUB, ROWS_PER_SUB)],
        )

    return pl.kernel(
        body,
        out_type=jax.ShapeDtypeStruct((2, NACC, AW), jnp.float32),
        mesh=mesh,
        compiler_params=pltpu.CompilerParams(use_tc_tiling_on_sc=False),
        scratch_types=[
            pltpu.VMEM((CS * EB,), jnp.int32),
            pltpu.VMEM((CS * EB,), jnp.int32),
            pltpu.VMEM((CS * EB,), jnp.int32),
            pltpu.VMEM((EB,), jnp.int32),
            pltpu.VMEM((EB,), jnp.int32),
            pltpu.VMEM((EB,), jnp.int32),
            pltpu.VMEM((EB,), jnp.int32),
            pltpu.VMEM((EB,), jnp.int32),
            pltpu.VMEM((EB,), jnp.int32),
            pltpu.VMEM((EB, 2 * 64), jnp.float32),
            pltpu.VMEM((EB, 2 * 64), jnp.float32),
            pltpu.VMEM((EB, 64), jnp.float32),
            pltpu.VMEM((EB, 64), jnp.float32),
            pltpu.VMEM((EB, AW), jnp.float32),
            pltpu.VMEM((EB, AW), jnp.float32),
            pltpu.VMEM_SHARED((NACC, AW), jnp.float32),
            pltpu.SemaphoreType.DMA,
            pltpu.SemaphoreType.DMA,
            pltpu.SemaphoreType.DMA,
            pltpu.SemaphoreType.DMA,
            pltpu.SemaphoreType.DMA,
            pltpu.SemaphoreType.DMA,
        ],
    )


_sc_writes = _make_sc_agg(0, 0)
_sc_cites = _make_sc_agg(1, 0)
_sc_rev = _make_sc_agg(2, 1)


def _blockdiag(mats):
    z = jnp.zeros((D, D), jnp.float32)
    for h in range(H):
        z = z.at[h * DH:(h + 1) * DH, h * DH:(h + 1) * DH].set(mats[h])
    return z


def _halves(w, b):
    w2 = w.shape[1]
    return (w.reshape(D, 2, w2 // 2).transpose(1, 0, 2),
            b.reshape(2, w2 // 2))


def _kv_halves(wk, bk_, wv, bv_):
    wh = [jnp.concatenate([wk[:, c * 64:(c + 1) * 64],
                           wv[:, c * 64:(c + 1) * 64]], axis=1) for c in (0, 1)]
    bh = [jnp.concatenate([bk_[c * 64:(c + 1) * 64],
                           bv_[c * 64:(c + 1) * 64]]) for c in (0, 1)]
    return jnp.stack(wh), jnp.stack(bh)


def kernel(x_paper, x_author, ei_writes, ei_cites, ei_rev, lin_in_W, lin_in_b,
           Wk, bk, Wq, bq, Wv, bv, Wa, ba, a_rel, m_rel, p_rel, skip):
    f32 = jnp.float32
    x_paper = x_paper.astype(f32)
    x_author = x_author.astype(f32)

    def pad_edges(ei):
        srcv = ei[0].astype(jnp.int32)
        dstv = ei[1].astype(jnp.int32)
        zpad = jnp.zeros((E_PAD - E,), jnp.int32)
        return (
            jnp.concatenate([srcv, zpad]),
            jnp.concatenate([dstv, zpad]),
            jnp.concatenate([dstv, jnp.full((E_PAD - E,), N, jnp.int32)]),
        )

    sw, gw, tw = pad_edges(ei_writes)
    sc_, gc, tc_ = pad_edges(ei_cites)
    sr, gr, tr = pad_edges(ei_rev)
    src_flat = jnp.concatenate([sw, sc_, sr])
    dstg_flat = jnp.concatenate([gw, gc, gr])
    dsts_flat = jnp.concatenate([tw, tc_, tr])

    xs = _lin_relu(
        jnp.stack([x_paper, x_author]),
        lin_in_W.astype(f32),
        lin_in_b.astype(f32),
    )
    xp, xa = xs[0], xs[1]

    scale = 1.0 / math.sqrt(DH)
    rel_src = (1, 0, 0)

    for l in range(L):
        wka, bka, wvm, bvm = [], [], [], []
        for r in range(3):
            st = rel_src[r]
            ablk = _blockdiag(a_rel[l, r] * (p_rel[l, r][:, None, None] * scale))
            mblk = _blockdiag(m_rel[l, r])
            wka.append(Wk[l, st] @ ablk)
            bka.append(bk[l, st] @ ablk)
            wvm.append(Wv[l, st] @ mblk)
            bvm.append(bv[l, st] @ mblk)

        qw_p, qb_p = _halves(Wq[l, 0], bq[l, 0])
        qw_a, qb_a = _halves(Wq[l, 1], bq[l, 1])
        kvw_c, kvb_c = _kv_halves(wka[1], bka[1], wvm[1], bvm[1])
        kvw_r, kvb_r = _kv_halves(wka[2], bka[2], wvm[2], bvm[2])
        kvw_w, kvb_w = _kv_halves(wka[0], bka[0], wvm[0], bvm[0])

        q_p = _proj(xp, qw_p[None], qb_p[None], 1, 64)
        q_a = _proj(xa, qw_a[None], qb_a[None], 1, 64)
        kv_p = _proj(xp, jnp.stack([kvw_c, kvw_r]), jnp.stack([kvb_c, kvb_r]),
                     2, 128)
        kv_a = _proj(xa, kvw_w[None], kvb_w[None], 1, 128)

        zro = jnp.zeros((NACC, AW), f32)
        acc_w = _sc_writes(src_flat, dstg_flat, dsts_flat, q_p, kv_a, zro)
        acc_c = _sc_cites(src_flat, dstg_flat, dsts_flat, q_p, kv_p, zro)
        acc_r = _sc_rev(src_flat, dstg_flat, dsts_flat, q_a, kv_p, zro)

        beta_p = jax.nn.sigmoid(skip[l, 0]).astype(f32)
        beta_a = jax.nn.sigmoid(skip[l, 1]).astype(f32)
        xp = _post([acc_w[:, :N], acc_c[:, :N]], xp, Wa[l, 0], ba[l, 0], beta_p)
        xa = _post([acc_r[:, :N]], xa, Wa[l, 1], ba[l, 1], beta_a)

    return xp, xa
